# baseline (speedup 1.0000x reference)
import functools

import jax
import jax.numpy as jnp
from jax import lax
from jax.experimental import pallas as pl
from jax.experimental.pallas import tpu as pltpu
from jax.experimental.pallas import tpu_sc as plsc

_NC = 2
_NS = 16
_NW = _NC * _NS
_B = 128
_SLOPE = 0.01


def _logits_body(x_ref, ai_ref, aj_ref, ei_ref, ej_ref):
    xb = x_ref[...]
    ei_ref[...] = jnp.sum(xb * ai_ref[...], axis=1, keepdims=True)
    ej_ref[...] = jnp.sum(xb * aj_ref[...], axis=1, keepdims=True)


def _logits(x, a_i, a_j, bn):
    n, h = x.shape
    grid = n // bn
    return pl.pallas_call(
        _logits_body,
        grid=(grid,),
        in_specs=[
            pl.BlockSpec((bn, h), lambda i: (i, 0)),
            pl.BlockSpec((1, h), lambda i: (0, 0)),
            pl.BlockSpec((1, h), lambda i: (0, 0)),
        ],
        out_specs=[
            pl.BlockSpec((bn, 1), lambda i: (i, 0)),
            pl.BlockSpec((bn, 1), lambda i: (i, 0)),
        ],
        out_shape=[
            jax.ShapeDtypeStruct((n, 1), jnp.float32),
            jax.ShapeDtypeStruct((n, 1), jnp.float32),
        ],
    )(x, a_i, a_j)


def _tile_rows(n, s):
    per = (n // _NS) // 8 * 8
    base = per * s
    last = n - per * (_NS - 1)
    return base, per, last


def _sc_edges_body(n, e, h, npad, x_hbm, idx3_hbm, eij_hbm,
                   acc_out, den_out,
                   idx0, idx1, ei0, ei1, ej0, ej1, w0, w1, rows0, rows1,
                   acc_sh, den_sh,
                   gsem0, gsem1, ssem0, ssem1, dsem0, dsem1):
    c = lax.axis_index("c")
    s = lax.axis_index("s")
    wid = s * _NC + c
    n_chunks = e // _B
    dpt = npad // _NS
    groups = _B // 16
    idxb = (idx0, idx1)
    eib = (ei0, ei1)
    ejb = (ej0, ej1)
    wb = (w0, w1)
    rowsb = (rows0, rows1)
    gsem = (gsem0, gsem1)
    ssem = (ssem0, ssem1)
    dsem = (dsem0, dsem1)

    def _start(k, b):
        cid = wid + k * _NW

        @pl.when(cid < n_chunks)
        def _():
            pltpu.sync_copy(idx3_hbm.at[cid], idxb[b])
            pltpu.async_copy(eij_hbm.at[idxb[b].at[1]], eib[b], gsem[b])
            pltpu.async_copy(eij_hbm.at[idxb[b].at[2]], ejb[b], gsem[b])
            pltpu.async_copy(x_hbm.at[idxb[b].at[0]], rowsb[b], gsem[b])

    def _wait_gathers(b):
        pltpu.make_async_copy(eij_hbm.at[idxb[b].at[1]], eib[b],
                              gsem[b]).wait()
        pltpu.make_async_copy(eij_hbm.at[idxb[b].at[2]], ejb[b],
                              gsem[b]).wait()
        pltpu.make_async_copy(x_hbm.at[idxb[b].at[0]], rowsb[b],
                              gsem[b]).wait()

    def _wait_scatters(b):
        pltpu.make_async_copy(
            rowsb[b], acc_sh.at[idxb[b].at[1]], ssem[b]).wait()
        pltpu.make_async_copy(
            wb[b].at[0], den_sh.at[idxb[b].at[1]], dsem[b]).wait()

    _start(0, 0)

    for g in range(_B // 16):
        w0[0, pl.ds(g * 16, 16)] = jnp.zeros((16,), jnp.float32)
    for i in range(dpt // _B):
        pltpu.sync_copy(w0.at[0], den_sh.at[pl.ds(s * dpt + i * _B, _B)])

    abase, aper, alast = _tile_rows(n, s)

    def _zero_acc(n_rows):
        def _z(r, _):
            for hh in range(h // 16):
                rows1[r, pl.ds(hh * 16, 16)] = jnp.zeros((16,), jnp.float32)
            return _
        lax.fori_loop(0, _B, _z, None)
        full, rem = n_rows // _B, n_rows % _B
        for i in range(full):
            pltpu.sync_copy(rows1, acc_sh.at[pl.ds(abase + i * _B, _B)])
        if rem:
            pltpu.sync_copy(rows1.at[pl.ds(0, rem)],
                            acc_sh.at[pl.ds(abase + full * _B, rem)])

    @pl.when(s == _NS - 1)
    def _():
        _zero_acc(alast)

    @pl.when(s < _NS - 1)
    def _():
        _zero_acc(aper)

    plsc.subcore_barrier()

    n_iter = (n_chunks + _NW - 1) // _NW

    def _finish(k, b):
        cid = wid + k * _NW

        @pl.when(cid < n_chunks)
        def _():
            _wait_gathers(b)
            for g in range(groups):
                sl = pl.ds(g * 16, 16)
                ev = eib[b][sl] + ejb[b][sl]
                ev = jnp.where(ev >= 0.0, ev, ev * _SLOPE)
                w = jnp.where(idxb[b][0, sl] != idxb[b][1, sl],
                              jnp.exp(ev), 0.0)
                wb[b][0, sl] = w

            @pl.when(k >= 1)
            def _():
                _wait_scatters(1 - b)
            _start(k + 1, 1 - b)

            def _scale(g, _c):
                w16 = wb[b][0, pl.ds(g * 16, 16)]
                for j in range(16):
                    wv = w16[j]
                    r = g * 16 + j
                    for hh in range(h // 16):
                        slh = pl.ds(hh * 16, 16)
                        rowsb[b][r, slh] = rowsb[b][r, slh] * wv
                return _c
            lax.fori_loop(0, groups, _scale, None)

            pltpu.async_copy(rowsb[b], acc_sh.at[idxb[b].at[1]], ssem[b],
                             add=True)
            pltpu.async_copy(wb[b].at[0], den_sh.at[idxb[b].at[1]], dsem[b],
                             add=True)

    def _pair(k2, _):
        k = k2 * 2
        _finish(k, 0)
        _finish(k + 1, 1)
        return _

    lax.fori_loop(0, n_iter // 2, _pair, None)
    for k in range(n_iter // 2 * 2, n_iter):
        _finish(k, k % 2)

    for kk in (n_iter - 2, n_iter - 1):
        c0 = wid + kk * _NW
        c1 = wid + (kk + 1) * _NW

        @pl.when((c0 < n_chunks) & (c1 >= n_chunks))
        def _(kk=kk):
            _wait_scatters(kk % 2)

    plsc.subcore_barrier()

    @pl.when(s == _NS - 1)
    def _():
        pltpu.sync_copy(acc_sh.at[pl.ds(abase, alast)],
                        acc_out.at[c, pl.ds(abase, alast)])

    @pl.when(s < _NS - 1)
    def _():
        pltpu.sync_copy(acc_sh.at[pl.ds(abase, aper)],
                        acc_out.at[c, pl.ds(abase, aper)])

    pltpu.sync_copy(den_sh.at[pl.ds(s * dpt, dpt)], den_out.at[c, s])


def _sc_edges(x, idx3, eij, npad):
    n, h = x.shape
    e = idx3.shape[0] * _B
    mesh = plsc.VectorSubcoreMesh(core_axis_name="c", subcore_axis_name="s")
    kfn = pl.kernel(
        functools.partial(_sc_edges_body, n, e, h, npad),
        out_type=[
            jax.ShapeDtypeStruct((_NC, n, h), jnp.float32),
            jax.ShapeDtypeStruct((_NC, _NS, npad // _NS), jnp.float32),
        ],
        mesh=mesh,
        scratch_types=[
            pltpu.VMEM((3, _B), jnp.int32),
            pltpu.VMEM((3, _B), jnp.int32),
            pltpu.VMEM((_B,), jnp.float32),
            pltpu.VMEM((_B,), jnp.float32),
            pltpu.VMEM((_B,), jnp.float32),
            pltpu.VMEM((_B,), jnp.float32),
            pltpu.VMEM((1, _B), jnp.float32),
            pltpu.VMEM((1, _B), jnp.float32),
            pltpu.VMEM((_B, h), jnp.float32),
            pltpu.VMEM((_B, h), jnp.float32),
            pltpu.VMEM_SHARED((n, h), jnp.float32),
            pltpu.VMEM_SHARED((npad,), jnp.float32),
        ] + [pltpu.SemaphoreType.DMA] * 6,
        compiler_params=pltpu.CompilerParams(needs_layout_passes=False),
    )
    return kfn(x, idx3, eij)


def _finalize_body(acc0_ref, acc1_ref, den0_ref, den1_ref, ei_ref, ej_ref,
                   x_ref, out_ref):
    eself = ei_ref[...] + ej_ref[...]
    eself = jnp.where(eself >= 0.0, eself, eself * _SLOPE)
    wself = jnp.exp(eself)
    den = den0_ref[...] + den1_ref[...] + wself
    num = acc0_ref[0] + acc1_ref[0] + wself * x_ref[...]
    out_ref[...] = jnp.maximum(num / den, 0.0)


def _finalize(accp, den0, den1, ei, ej, x, bn):
    n, h = x.shape
    grid = n // bn
    col = pl.BlockSpec((bn, 1), lambda i: (i, 0))
    mat = pl.BlockSpec((bn, h), lambda i: (i, 0))
    acc0_spec = pl.BlockSpec((1, bn, h), lambda i: (0, i, 0))
    acc1_spec = pl.BlockSpec((1, bn, h), lambda i: (1, i, 0))
    return pl.pallas_call(
        _finalize_body,
        grid=(grid,),
        in_specs=[acc0_spec, acc1_spec, col, col, col, col, mat],
        out_specs=mat,
        out_shape=jax.ShapeDtypeStruct((n, h), jnp.float32),
    )(accp, accp, den0, den1, ei, ej, x)


def kernel(x, edge_index, a_i, a_j):
    n, h = x.shape
    e = edge_index.shape[1]
    assert n % 8 == 0 and h % 16 == 0 and e % _B == 0
    npad = ((n + _NS * 16 - 1) // (_NS * 16)) * (_NS * 16)
    eidx = edge_index.astype(jnp.int32)
    nc = e // _B
    src = eidx[0].reshape(nc, _B)
    dst = eidx[1].reshape(nc, _B)
    idx3 = jnp.stack([src, dst, src + n], axis=1)
    ei2, ej2 = _logits(x, a_i, a_j, bn=1000)
    eij = jnp.concatenate([ei2[:, 0], ej2[:, 0]])
    accp, den3 = _sc_edges(x, idx3, eij, npad)
    den = den3.reshape(_NC, npad)[:, :n, None]
    out = _finalize(accp, den[0], den[1], ei2, ej2, x, bn=1000)
    return out


# speedup vs baseline: 1.0608x; 1.0608x over previous
import functools

import jax
import jax.numpy as jnp
from jax import lax
from jax.experimental import pallas as pl
from jax.experimental.pallas import tpu as pltpu
from jax.experimental.pallas import tpu_sc as plsc

_NC = 2
_NS = 16
_NW = _NC * _NS
_B = 128
_SLOPE = 0.01


def _logits_body(x_ref, ai_ref, aj_ref, ei_ref, ej_ref):
    xb = x_ref[...]
    ei_ref[...] = jnp.sum(xb * ai_ref[...], axis=1, keepdims=True)
    ej_ref[...] = jnp.sum(xb * aj_ref[...], axis=1, keepdims=True)


def _logits(x, a_i, a_j, bn):
    n, h = x.shape
    grid = n // bn
    return pl.pallas_call(
        _logits_body,
        grid=(grid,),
        in_specs=[
            pl.BlockSpec((bn, h), lambda i: (i, 0)),
            pl.BlockSpec((1, h), lambda i: (0, 0)),
            pl.BlockSpec((1, h), lambda i: (0, 0)),
        ],
        out_specs=[
            pl.BlockSpec((bn, 1), lambda i: (i, 0)),
            pl.BlockSpec((bn, 1), lambda i: (i, 0)),
        ],
        out_shape=[
            jax.ShapeDtypeStruct((n, 1), jnp.float32),
            jax.ShapeDtypeStruct((n, 1), jnp.float32),
        ],
    )(x, a_i, a_j)


def _tile_rows(n, s):
    per = (n // _NS) // 8 * 8
    base = per * s
    last = n - per * (_NS - 1)
    return base, per, last


def _sc_edges_body(n, e, h, npad, x_hbm, eidx_hbm, ei_hbm, ej_hbm,
                   acc_out, den_out,
                   idx0, idx1, ei0, ei1, ej0, ej1, w0, w1, rows0, rows1,
                   acc_sh, den_sh,
                   gsem0, gsem1, ssem0, ssem1, dsem0, dsem1):
    c = lax.axis_index("c")
    s = lax.axis_index("s")
    wid = s * _NC + c
    n_chunks = e // _B
    dpt = npad // _NS
    groups = _B // 16
    idxb = (idx0, idx1)
    eib = (ei0, ei1)
    ejb = (ej0, ej1)
    wb = (w0, w1)
    rowsb = (rows0, rows1)
    gsem = (gsem0, gsem1)
    ssem = (ssem0, ssem1)
    dsem = (dsem0, dsem1)

    def _start(k, b):
        cid = wid + k * _NW

        @pl.when(cid < n_chunks)
        def _():
            off = cid * _B
            pltpu.sync_copy(eidx_hbm.at[pl.ds(0, 2), pl.ds(off, _B)], idxb[b])
            pltpu.async_copy(ei_hbm.at[idxb[b].at[1]], eib[b], gsem[b])
            pltpu.async_copy(ej_hbm.at[idxb[b].at[0]], ejb[b], gsem[b])
            pltpu.async_copy(x_hbm.at[idxb[b].at[0]], rowsb[b], gsem[b])

    def _wait_gathers(b):
        pltpu.make_async_copy(ei_hbm.at[idxb[b].at[1]], eib[b],
                              gsem[b]).wait()
        pltpu.make_async_copy(ej_hbm.at[idxb[b].at[0]], ejb[b],
                              gsem[b]).wait()
        pltpu.make_async_copy(x_hbm.at[idxb[b].at[0]], rowsb[b],
                              gsem[b]).wait()

    def _wait_scatters(b):
        pltpu.make_async_copy(
            rowsb[b], acc_sh.at[idxb[b].at[1]], ssem[b]).wait()
        pltpu.make_async_copy(
            wb[b].at[0], den_sh.at[idxb[b].at[1]], dsem[b]).wait()

    _start(0, 0)

    for g in range(_B // 16):
        w0[0, pl.ds(g * 16, 16)] = jnp.zeros((16,), jnp.float32)
    for i in range(dpt // _B):
        pltpu.sync_copy(w0.at[0], den_sh.at[pl.ds(s * dpt + i * _B, _B)])

    abase, aper, alast = _tile_rows(n, s)

    def _zero_acc(n_rows):
        def _z(r, _):
            for hh in range(h // 16):
                rows1[r, pl.ds(hh * 16, 16)] = jnp.zeros((16,), jnp.float32)
            return _
        lax.fori_loop(0, _B, _z, None)
        full, rem = n_rows // _B, n_rows % _B
        for i in range(full):
            pltpu.sync_copy(rows1, acc_sh.at[pl.ds(abase + i * _B, _B)])
        if rem:
            pltpu.sync_copy(rows1.at[pl.ds(0, rem)],
                            acc_sh.at[pl.ds(abase + full * _B, rem)])

    @pl.when(s == _NS - 1)
    def _():
        _zero_acc(alast)

    @pl.when(s < _NS - 1)
    def _():
        _zero_acc(aper)

    plsc.subcore_barrier()

    n_iter = (n_chunks + _NW - 1) // _NW

    def _finish(k, b):
        cid = wid + k * _NW

        @pl.when(cid < n_chunks)
        def _():
            _wait_gathers(b)
            for g in range(groups):
                sl = pl.ds(g * 16, 16)
                ev = eib[b][sl] + ejb[b][sl]
                ev = jnp.where(ev >= 0.0, ev, ev * _SLOPE)
                w = jnp.where(idxb[b][0, sl] != idxb[b][1, sl],
                              jnp.exp(ev), 0.0)
                wb[b][0, sl] = w

            @pl.when(k >= 1)
            def _():
                _wait_scatters(1 - b)
            _start(k + 1, 1 - b)

            def _scale(g, _c):
                w16 = wb[b][0, pl.ds(g * 16, 16)]
                for j in range(16):
                    wv = w16[j]
                    r = g * 16 + j
                    for hh in range(h // 16):
                        slh = pl.ds(hh * 16, 16)
                        rowsb[b][r, slh] = rowsb[b][r, slh] * wv
                return _c
            lax.fori_loop(0, groups, _scale, None)

            pltpu.async_copy(rowsb[b], acc_sh.at[idxb[b].at[1]], ssem[b],
                             add=True)
            pltpu.async_copy(wb[b].at[0], den_sh.at[idxb[b].at[1]], dsem[b],
                             add=True)

    def _pair(k2, _):
        k = k2 * 2
        _finish(k, 0)
        _finish(k + 1, 1)
        return _

    lax.fori_loop(0, n_iter // 2, _pair, None)
    for k in range(n_iter // 2 * 2, n_iter):
        _finish(k, k % 2)

    for kk in (n_iter - 2, n_iter - 1):
        c0 = wid + kk * _NW
        c1 = wid + (kk + 1) * _NW

        @pl.when((c0 < n_chunks) & (c1 >= n_chunks))
        def _(kk=kk):
            _wait_scatters(kk % 2)

    plsc.subcore_barrier()

    @pl.when(s == _NS - 1)
    def _():
        pltpu.sync_copy(acc_sh.at[pl.ds(abase, alast)],
                        acc_out.at[c, pl.ds(abase, alast)])

    @pl.when(s < _NS - 1)
    def _():
        pltpu.sync_copy(acc_sh.at[pl.ds(abase, aper)],
                        acc_out.at[c, pl.ds(abase, aper)])

    pltpu.sync_copy(den_sh.at[pl.ds(s * dpt, dpt)], den_out.at[c, s])


def _sc_edges(x, eidx, ei, ej, npad):
    n, h = x.shape
    e = eidx.shape[1]
    mesh = plsc.VectorSubcoreMesh(core_axis_name="c", subcore_axis_name="s")
    kfn = pl.kernel(
        functools.partial(_sc_edges_body, n, e, h, npad),
        out_type=[
            jax.ShapeDtypeStruct((_NC, n, h), jnp.float32),
            jax.ShapeDtypeStruct((_NC, _NS, npad // _NS), jnp.float32),
        ],
        mesh=mesh,
        scratch_types=[
            pltpu.VMEM((2, _B), jnp.int32),
            pltpu.VMEM((2, _B), jnp.int32),
            pltpu.VMEM((_B,), jnp.float32),
            pltpu.VMEM((_B,), jnp.float32),
            pltpu.VMEM((_B,), jnp.float32),
            pltpu.VMEM((_B,), jnp.float32),
            pltpu.VMEM((1, _B), jnp.float32),
            pltpu.VMEM((1, _B), jnp.float32),
            pltpu.VMEM((_B, h), jnp.float32),
            pltpu.VMEM((_B, h), jnp.float32),
            pltpu.VMEM_SHARED((n, h), jnp.float32),
            pltpu.VMEM_SHARED((npad,), jnp.float32),
        ] + [pltpu.SemaphoreType.DMA] * 6,
        compiler_params=pltpu.CompilerParams(needs_layout_passes=False),
    )
    return kfn(x, eidx, ei, ej)


def _finalize_body(acc0_ref, acc1_ref, den0_ref, den1_ref, ei_ref, ej_ref,
                   x_ref, out_ref):
    eself = ei_ref[...] + ej_ref[...]
    eself = jnp.where(eself >= 0.0, eself, eself * _SLOPE)
    wself = jnp.exp(eself)
    den = den0_ref[...] + den1_ref[...] + wself
    num = acc0_ref[0] + acc1_ref[0] + wself * x_ref[...]
    out_ref[...] = jnp.maximum(num / den, 0.0)


def _finalize(accp, den0, den1, ei, ej, x, bn):
    n, h = x.shape
    grid = n // bn
    col = pl.BlockSpec((bn, 1), lambda i: (i, 0))
    mat = pl.BlockSpec((bn, h), lambda i: (i, 0))
    acc0_spec = pl.BlockSpec((1, bn, h), lambda i: (0, i, 0))
    acc1_spec = pl.BlockSpec((1, bn, h), lambda i: (1, i, 0))
    return pl.pallas_call(
        _finalize_body,
        grid=(grid,),
        in_specs=[acc0_spec, acc1_spec, col, col, col, col, mat],
        out_specs=mat,
        out_shape=jax.ShapeDtypeStruct((n, h), jnp.float32),
    )(accp, accp, den0, den1, ei, ej, x)


def kernel(x, edge_index, a_i, a_j):
    n, h = x.shape
    e = edge_index.shape[1]
    assert n % 8 == 0 and h % 16 == 0 and e % _B == 0
    npad = ((n + _NS * 16 - 1) // (_NS * 16)) * (_NS * 16)
    eidx = edge_index.astype(jnp.int32)
    ei2, ej2 = _logits(x, a_i, a_j, bn=1000)
    accp, den3 = _sc_edges(x, eidx, ei2[:, 0], ej2[:, 0], npad)
    den = den3.reshape(_NC, npad)[:, :n, None]
    out = _finalize(accp, den[0], den[1], ei2, ej2, x, bn=1000)
    return out
